# Initial kernel scaffold; baseline (speedup 1.0000x reference)
import jax
import jax.numpy as jnp
from jax.experimental import pallas as pl


def kernel(user_emb, item_emb, u_trust, guided_item):
    raise NotImplementedError("write your pallas kernel here")


# speedup vs baseline: 8.8161x; 8.8161x over previous
import functools

import jax
import jax.numpy as jnp
from jax import lax
from jax.experimental import pallas as pl
from jax.experimental.pallas import tpu as pltpu
from jax.experimental.pallas import tpu_sc as plsc

NUM_CORES = 2
NUM_SUBCORES = 16
NW = NUM_CORES * NUM_SUBCORES
L = 16
D = 32
C = 512


def _sigmoid(x):
    return 1.0 / (1.0 + jnp.exp(-x))


def _make_kernel(U, I, E):
    assert E % C == 0
    n_chunks = E // C
    u_slice = ((U + NUM_SUBCORES - 1) // NUM_SUBCORES + 15) // 16 * 16
    u_pad = u_slice * NUM_SUBCORES

    mesh = plsc.VectorSubcoreMesh(
        core_axis_name="c", subcore_axis_name="s",
        num_cores=NUM_CORES, num_subcores=NUM_SUBCORES)

    @functools.partial(
        pl.kernel,
        out_type=jax.ShapeDtypeStruct((E,), jnp.float32),
        mesh=mesh,
        scratch_types=[
            pltpu.VMEM((C,), jnp.int32),
            pltpu.VMEM((C,), jnp.int32),
            pltpu.VMEM((C, D), jnp.float32),
            pltpu.VMEM((C, D), jnp.float32),
            pltpu.VMEM((C,), jnp.float32),
            pltpu.VMEM((C,), jnp.float32),
            pltpu.VMEM((C,), jnp.float32),
            pltpu.VMEM((u_slice,), jnp.float32),
            pltpu.VMEM((1, D), jnp.float32),
            pltpu.VMEM((D, L), jnp.float32),
            pltpu.VMEM((1,), jnp.int32),
            pltpu.VMEM_SHARED((u_pad,), jnp.float32),
            pltpu.SemaphoreType.DMA,
            pltpu.SemaphoreType.DMA,
        ],
        compiler_params=pltpu.CompilerParams(
            needs_layout_passes=False, use_tc_tiling_on_sc=False),
    )
    def k(user_hbm, item_hbm, src_hbm, dst_hbm, gidx_hbm, out_hbm,
          idx_s, idx_d, rows_s, rows_d, deg_e, out_e, ones_v, zeros_v,
          g_row, g_bcast, g_idx_v, deg_sh, sem0, sem1):
        cid = lax.axis_index("c")
        sid = lax.axis_index("s")
        wid = sid * NUM_CORES + cid

        zero16 = jnp.zeros((L,), jnp.float32)
        one16 = jnp.ones((L,), jnp.float32)

        def fill_zeros(i, carry):
            zeros_v[pl.ds(i * L, L)] = zero16
            return carry

        lax.fori_loop(0, u_slice // L, fill_zeros, 0)

        def fill_ones(i, carry):
            ones_v[pl.ds(i * L, L)] = one16
            return carry

        lax.fori_loop(0, C // L, fill_ones, 0)

        pltpu.sync_copy(gidx_hbm.at[pl.ds(0, 1)], g_idx_v)
        pltpu.async_copy(item_hbm.at[g_idx_v], g_row, sem0).wait()
        g_vec0 = g_row[0, pl.ds(0, L)]
        g_vec1 = g_row[0, pl.ds(L, L)]
        for j in range(D):
            half = g_vec0 if j < L else g_vec1
            jj = jnp.full((L,), j % L, jnp.int32)
            g_bcast[j] = jnp.take_along_axis(half, jj, axis=0)

        pltpu.sync_copy(zeros_v, deg_sh.at[pl.ds(sid * u_slice, u_slice)])
        plsc.subcore_barrier()

        def deg_body(i, carry):
            base = (sid + i * NUM_SUBCORES) * C
            pltpu.sync_copy(dst_hbm.at[pl.ds(base, C)], idx_d)
            pltpu.sync_copy(ones_v, deg_sh.at[idx_d], add=True)
            return carry

        n_deg = (n_chunks - sid + NUM_SUBCORES - 1) // NUM_SUBCORES
        lax.fori_loop(0, n_deg, deg_body, 0)
        plsc.subcore_barrier()

        iota16 = lax.iota(jnp.int32, L)

        def edge_body(i, carry):
            base = (wid + i * NW) * C
            pltpu.sync_copy(src_hbm.at[pl.ds(base, C)], idx_s)
            pltpu.sync_copy(dst_hbm.at[pl.ds(base, C)], idx_d)
            cp_s = pltpu.async_copy(user_hbm.at[idx_s], rows_s, sem0)
            cp_d = pltpu.async_copy(user_hbm.at[idx_d], rows_d, sem1)
            pltpu.sync_copy(deg_sh.at[idx_d], deg_e)
            cp_s.wait()
            cp_d.wait()

            def group(g, gcarry):
                eidx = iota16 + g * L
                acc_st = zero16
                acc_pf = zero16
                for j in range(D):
                    jj = jnp.full((L,), j, jnp.int32)
                    s_col = plsc.load_gather(rows_s, [eidx, jj])
                    t_col = plsc.load_gather(rows_d, [eidx, jj])
                    acc_st = acc_st + s_col * t_col
                    acc_pf = acc_pf + s_col * g_bcast[j]
                deg = deg_e[pl.ds(g * L, L)]
                att = _sigmoid(acc_st) / deg
                out_e[pl.ds(g * L, L)] = _sigmoid(att * acc_pf) - 0.5
                return gcarry

            lax.fori_loop(0, C // L, group, 0)
            pltpu.sync_copy(out_e, out_hbm.at[pl.ds(base, C)])
            return carry

        n_edge = (n_chunks - wid + NW - 1) // NW
        lax.fori_loop(0, n_edge, edge_body, 0)

    return k


def kernel(user_emb, item_emb, u_trust, guided_item):
    U = user_emb.shape[0]
    I = item_emb.shape[0]
    E = u_trust.shape[1]
    src = u_trust[0].astype(jnp.int32)
    dst = u_trust[1].astype(jnp.int32)
    gidx = jnp.full((8,), guided_item, jnp.int32)
    out = _make_kernel(U, I, E)(
        user_emb.astype(jnp.float32), item_emb.astype(jnp.float32),
        src, dst, gidx)
    return out.reshape(E, 1)


# speedup vs baseline: 18.8521x; 2.1384x over previous
import functools

import jax
import jax.numpy as jnp
from jax import lax
from jax.experimental import pallas as pl
from jax.experimental.pallas import tpu as pltpu
from jax.experimental.pallas import tpu_sc as plsc

NUM_CORES = 2
NUM_SUBCORES = 16
NW = NUM_CORES * NUM_SUBCORES
L = 16
D = 32
C = 512


def _sigmoid(x):
    return 1.0 / (1.0 + jnp.exp(-x))


def _make_kernel(U, I, E):
    assert E % C == 0
    n_chunks = E // C
    u_slice = ((U + NUM_SUBCORES - 1) // NUM_SUBCORES + 15) // 16 * 16
    u_pad = u_slice * NUM_SUBCORES

    mesh = plsc.VectorSubcoreMesh(
        core_axis_name="c", subcore_axis_name="s",
        num_cores=NUM_CORES, num_subcores=NUM_SUBCORES)

    @functools.partial(
        pl.kernel,
        out_type=jax.ShapeDtypeStruct((E,), jnp.float32),
        mesh=mesh,
        scratch_types=[
            pltpu.VMEM((C,), jnp.int32),
            pltpu.VMEM((C,), jnp.int32),
            pltpu.VMEM((C, D), jnp.float32),
            pltpu.VMEM((C, D), jnp.float32),
            pltpu.VMEM((C,), jnp.float32),
            pltpu.VMEM((C,), jnp.float32),
            pltpu.VMEM((C,), jnp.float32),
            pltpu.VMEM((u_slice,), jnp.float32),
            pltpu.VMEM((1, D), jnp.float32),
            pltpu.VMEM((D, L), jnp.float32),
            pltpu.VMEM((1,), jnp.int32),
            pltpu.VMEM_SHARED((u_pad,), jnp.float32),
            pltpu.SemaphoreType.DMA,
            pltpu.SemaphoreType.DMA,
        ],
        compiler_params=pltpu.CompilerParams(
            needs_layout_passes=False, use_tc_tiling_on_sc=False),
    )
    def k(user_hbm, item_hbm, src_hbm, dst_hbm, gidx_hbm, out_hbm,
          idx_s, idx_d, rows_s, rows_d, deg_e, out_e, ones_v, zeros_v,
          g_row, g_bcast, g_idx_v, deg_sh, sem0, sem1):
        cid = lax.axis_index("c")
        sid = lax.axis_index("s")
        wid = sid * NUM_CORES + cid

        zero16 = jnp.zeros((L,), jnp.float32)
        one16 = jnp.ones((L,), jnp.float32)

        def fill_zeros(i, carry):
            zeros_v[pl.ds(i * L, L)] = zero16
            return carry

        lax.fori_loop(0, u_slice // L, fill_zeros, 0)

        def fill_ones(i, carry):
            ones_v[pl.ds(i * L, L)] = one16
            return carry

        lax.fori_loop(0, C // L, fill_ones, 0)

        pltpu.sync_copy(gidx_hbm.at[pl.ds(0, 1)], g_idx_v)
        pltpu.async_copy(item_hbm.at[g_idx_v], g_row, sem0).wait()
        g_vec0 = g_row[0, pl.ds(0, L)]
        g_vec1 = g_row[0, pl.ds(L, L)]
        iota16b = lax.iota(jnp.int32, L)
        for j in range(D):
            v = (iota16b + j) % D
            lo = jnp.take_along_axis(g_vec0, v % L, axis=0)
            hi = jnp.take_along_axis(g_vec1, v % L, axis=0)
            g_bcast[j] = jnp.where(v < L, lo, hi)

        pltpu.sync_copy(zeros_v, deg_sh.at[pl.ds(sid * u_slice, u_slice)])
        plsc.subcore_barrier()

        def deg_body(i, carry):
            base = (sid + i * NUM_SUBCORES) * C
            pltpu.sync_copy(dst_hbm.at[pl.ds(base, C)], idx_d)
            pltpu.sync_copy(ones_v, deg_sh.at[idx_d], add=True)
            return carry

        n_deg = (n_chunks - sid + NUM_SUBCORES - 1) // NUM_SUBCORES
        lax.fori_loop(0, n_deg, deg_body, 0)
        plsc.subcore_barrier()

        iota16 = lax.iota(jnp.int32, L)

        def edge_body(i, carry):
            base = (wid + i * NW) * C
            pltpu.sync_copy(src_hbm.at[pl.ds(base, C)], idx_s)
            pltpu.sync_copy(dst_hbm.at[pl.ds(base, C)], idx_d)
            cp_s = pltpu.async_copy(user_hbm.at[idx_s], rows_s, sem0)
            cp_d = pltpu.async_copy(user_hbm.at[idx_d], rows_d, sem1)
            pltpu.sync_copy(deg_sh.at[idx_d], deg_e)
            cp_s.wait()
            cp_d.wait()

            def group(g, gcarry):
                eidx = iota16 + g * L
                acc_st = zero16
                acc_pf = zero16
                for j in range(D):
                    jj = (iota16 + j) % D
                    s_col = plsc.load_gather(rows_s, [eidx, jj])
                    t_col = plsc.load_gather(rows_d, [eidx, jj])
                    acc_st = acc_st + s_col * t_col
                    acc_pf = acc_pf + s_col * g_bcast[j]
                deg = deg_e[pl.ds(g * L, L)]
                att = _sigmoid(acc_st) / deg
                out_e[pl.ds(g * L, L)] = _sigmoid(att * acc_pf) - 0.5
                return gcarry

            lax.fori_loop(0, C // L, group, 0)
            pltpu.sync_copy(out_e, out_hbm.at[pl.ds(base, C)])
            return carry

        n_edge = (n_chunks - wid + NW - 1) // NW
        lax.fori_loop(0, n_edge, edge_body, 0)

    return k


def kernel(user_emb, item_emb, u_trust, guided_item):
    U = user_emb.shape[0]
    I = item_emb.shape[0]
    E = u_trust.shape[1]
    src = u_trust[0].astype(jnp.int32)
    dst = u_trust[1].astype(jnp.int32)
    gidx = jnp.full((8,), guided_item, jnp.int32)
    out = _make_kernel(U, I, E)(
        user_emb.astype(jnp.float32), item_emb.astype(jnp.float32),
        src, dst, gidx)
    return out.reshape(E, 1)


# speedup vs baseline: 20.5711x; 1.0912x over previous
import functools

import jax
import jax.numpy as jnp
from jax import lax
from jax.experimental import pallas as pl
from jax.experimental.pallas import tpu as pltpu
from jax.experimental.pallas import tpu_sc as plsc

NUM_CORES = 2
NUM_SUBCORES = 16
NW = NUM_CORES * NUM_SUBCORES
L = 16
D = 32
C = 512
PC = 448


def _sigmoid(x):
    return 1.0 / (1.0 + jnp.exp(-x))


def _make_kernel(U, I, E):
    assert E % C == 0
    n_chunks = E // C
    u_slice = ((U + NUM_SUBCORES - 1) // NUM_SUBCORES + PC - 1) // PC * PC
    n_pc = u_slice // PC
    u_pad = u_slice * NUM_SUBCORES

    mesh = plsc.VectorSubcoreMesh(
        core_axis_name="c", subcore_axis_name="s",
        num_cores=NUM_CORES, num_subcores=NUM_SUBCORES)

    @functools.partial(
        pl.kernel,
        out_type=jax.ShapeDtypeStruct((E,), jnp.float32),
        mesh=mesh,
        scratch_types=[
            pltpu.VMEM((C,), jnp.int32),
            pltpu.VMEM((C,), jnp.int32),
            pltpu.VMEM((C, D), jnp.float32),
            pltpu.VMEM((C, D), jnp.float32),
            pltpu.VMEM((C,), jnp.float32),
            pltpu.VMEM((C,), jnp.float32),
            pltpu.VMEM((C,), jnp.float32),
            pltpu.VMEM((C,), jnp.float32),
            pltpu.VMEM((u_slice,), jnp.float32),
            pltpu.VMEM((1, D), jnp.float32),
            pltpu.VMEM((D, L), jnp.float32),
            pltpu.VMEM((1,), jnp.int32),
            pltpu.VMEM_SHARED((u_pad,), jnp.float32),
            pltpu.VMEM_SHARED((u_pad,), jnp.float32),
            pltpu.SemaphoreType.DMA,
            pltpu.SemaphoreType.DMA,
        ],
        compiler_params=pltpu.CompilerParams(
            needs_layout_passes=False, use_tc_tiling_on_sc=False),
    )
    def k(user_hbm, item_hbm, src_hbm, dst_hbm, gidx_hbm, out_hbm,
          idx_s, idx_d, rows_s, rows_d, deg_e, p_e, out_e, ones_v, zeros_v,
          g_row, g_bcast, g_idx_v, deg_sh, p_sh, sem0, sem1):
        cid = lax.axis_index("c")
        sid = lax.axis_index("s")
        wid = sid * NUM_CORES + cid

        zero16 = jnp.zeros((L,), jnp.float32)
        iota16 = lax.iota(jnp.int32, L)
        one16 = jnp.ones((L,), jnp.float32)

        def fill_zeros(i, carry):
            zeros_v[pl.ds(i * L, L)] = zero16
            return carry

        lax.fori_loop(0, u_slice // L, fill_zeros, 0)

        def fill_ones(i, carry):
            ones_v[pl.ds(i * L, L)] = one16
            return carry

        lax.fori_loop(0, C // L, fill_ones, 0)

        pltpu.sync_copy(gidx_hbm.at[pl.ds(0, 1)], g_idx_v)
        pltpu.async_copy(item_hbm.at[g_idx_v], g_row, sem0).wait()
        g_vec0 = g_row[0, pl.ds(0, L)]
        g_vec1 = g_row[0, pl.ds(L, L)]
        iota16b = lax.iota(jnp.int32, L)
        for j in range(D):
            v = (iota16b + j) % D
            lo = jnp.take_along_axis(g_vec0, v % L, axis=0)
            hi = jnp.take_along_axis(g_vec1, v % L, axis=0)
            g_bcast[j] = jnp.where(v < L, lo, hi)

        pltpu.sync_copy(zeros_v, deg_sh.at[pl.ds(sid * u_slice, u_slice)])
        plsc.subcore_barrier()

        def deg_body(i, carry):
            base = (sid + i * NUM_SUBCORES) * C
            pltpu.sync_copy(dst_hbm.at[pl.ds(base, C)], idx_d)
            pltpu.sync_copy(ones_v, deg_sh.at[idx_d], add=True)
            return carry

        n_deg = (n_chunks - sid + NUM_SUBCORES - 1) // NUM_SUBCORES
        lax.fori_loop(0, n_deg, deg_body, 0)

        def p_body(q, carry):
            ubase = sid * u_slice + q * PC
            pltpu.sync_copy(user_hbm.at[pl.ds(ubase, PC)],
                            rows_s.at[pl.ds(0, PC)])

            def p_group(g, gcarry):
                eidx = iota16 + g * L
                acc = zero16
                for j in range(D):
                    jj = (iota16 + j) % D
                    col = plsc.load_gather(rows_s, [eidx, jj])
                    acc = acc + col * g_bcast[j]
                out_e[pl.ds(g * L, L)] = acc
                return gcarry
            lax.fori_loop(0, PC // L, p_group, 0)
            pltpu.sync_copy(out_e.at[pl.ds(0, PC)],
                            p_sh.at[pl.ds(ubase, PC)])
            return carry
        lax.fori_loop(0, n_pc, p_body, 0)
        plsc.subcore_barrier()


        def edge_body(i, carry):
            base = (wid + i * NW) * C
            pltpu.sync_copy(src_hbm.at[pl.ds(base, C)], idx_s)
            pltpu.sync_copy(dst_hbm.at[pl.ds(base, C)], idx_d)
            cp_s = pltpu.async_copy(user_hbm.at[idx_s], rows_s, sem0)
            cp_d = pltpu.async_copy(user_hbm.at[idx_d], rows_d, sem1)
            pltpu.sync_copy(deg_sh.at[idx_d], deg_e)
            pltpu.sync_copy(p_sh.at[idx_s], p_e)
            cp_s.wait()
            cp_d.wait()

            def group(g, gcarry):
                eidx = iota16 + g * L
                acc_st = zero16
                for j in range(D):
                    jj = (iota16 + j) % D
                    s_col = plsc.load_gather(rows_s, [eidx, jj])
                    t_col = plsc.load_gather(rows_d, [eidx, jj])
                    acc_st = acc_st + s_col * t_col
                deg = deg_e[pl.ds(g * L, L)]
                acc_pf = p_e[pl.ds(g * L, L)]
                att = _sigmoid(acc_st) / deg
                out_e[pl.ds(g * L, L)] = _sigmoid(att * acc_pf) - 0.5
                return gcarry

            lax.fori_loop(0, C // L, group, 0)
            pltpu.sync_copy(out_e, out_hbm.at[pl.ds(base, C)])
            return carry

        n_edge = (n_chunks - wid + NW - 1) // NW
        lax.fori_loop(0, n_edge, edge_body, 0)

    return k


def kernel(user_emb, item_emb, u_trust, guided_item):
    U = user_emb.shape[0]
    I = item_emb.shape[0]
    E = u_trust.shape[1]
    src = u_trust[0].astype(jnp.int32)
    dst = u_trust[1].astype(jnp.int32)
    gidx = jnp.full((8,), guided_item, jnp.int32)
    u_slice = ((U + NUM_SUBCORES - 1) // NUM_SUBCORES + PC - 1) // PC * PC
    u_pad = u_slice * NUM_SUBCORES
    user_padded = jnp.pad(user_emb.astype(jnp.float32),
                          ((0, u_pad - U), (0, 0)))
    out = _make_kernel(U, I, E)(
        user_padded, item_emb.astype(jnp.float32), src, dst, gidx)
    return out.reshape(E, 1)


# speedup vs baseline: 22.9978x; 1.1180x over previous
import functools

import jax
import jax.numpy as jnp
from jax import lax
from jax.experimental import pallas as pl
from jax.experimental.pallas import tpu as pltpu
from jax.experimental.pallas import tpu_sc as plsc

NUM_CORES = 2
NUM_SUBCORES = 16
NW = NUM_CORES * NUM_SUBCORES
L = 16
D = 32
C = 512
PC = 448


def _sigmoid(x):
    return 1.0 / (1.0 + jnp.exp(-x))


def _make_kernel(U, I, E):
    assert E % C == 0
    n_chunks = E // C
    u_slice = ((U + NUM_SUBCORES - 1) // NUM_SUBCORES + PC - 1) // PC * PC
    n_pc = u_slice // PC
    u_pad = u_slice * NUM_SUBCORES

    mesh = plsc.VectorSubcoreMesh(
        core_axis_name="c", subcore_axis_name="s",
        num_cores=NUM_CORES, num_subcores=NUM_SUBCORES)

    @functools.partial(
        pl.kernel,
        out_type=jax.ShapeDtypeStruct((E,), jnp.float32),
        mesh=mesh,
        scratch_types=[
            pltpu.VMEM((C,), jnp.int32),
            pltpu.VMEM((C,), jnp.int32),
            pltpu.VMEM((2 * C, D), jnp.float32),
            pltpu.VMEM((2 * C, D), jnp.float32),
            pltpu.VMEM((2 * C,), jnp.float32),
            pltpu.VMEM((2 * C,), jnp.float32),
            pltpu.VMEM((C,), jnp.float32),
            pltpu.VMEM((C,), jnp.float32),
            pltpu.VMEM((u_slice,), jnp.float32),
            pltpu.VMEM((1, D), jnp.float32),
            pltpu.VMEM((D, L), jnp.float32),
            pltpu.VMEM((1,), jnp.int32),
            pltpu.VMEM_SHARED((u_pad,), jnp.float32),
            pltpu.VMEM_SHARED((u_pad,), jnp.float32),
            pltpu.SemaphoreType.DMA,
            pltpu.SemaphoreType.DMA,
        ],
        compiler_params=pltpu.CompilerParams(
            needs_layout_passes=False, use_tc_tiling_on_sc=False),
    )
    def k(user_hbm, item_hbm, src_hbm, dst_hbm, gidx_hbm, out_hbm,
          idx_s, idx_d, rows_s, rows_d, deg_e, p_e, out_e, ones_v, zeros_v,
          g_row, g_bcast, g_idx_v, deg_sh, p_sh, sem0, sem1):
        cid = lax.axis_index("c")
        sid = lax.axis_index("s")
        wid = sid * NUM_CORES + cid

        zero16 = jnp.zeros((L,), jnp.float32)
        iota16 = lax.iota(jnp.int32, L)
        one16 = jnp.ones((L,), jnp.float32)

        def fill_zeros(i, carry):
            zeros_v[pl.ds(i * L, L)] = zero16
            return carry

        lax.fori_loop(0, u_slice // L, fill_zeros, 0)

        def fill_ones(i, carry):
            ones_v[pl.ds(i * L, L)] = one16
            return carry

        lax.fori_loop(0, C // L, fill_ones, 0)

        pltpu.sync_copy(gidx_hbm.at[pl.ds(0, 1)], g_idx_v)
        pltpu.async_copy(item_hbm.at[g_idx_v], g_row, sem0).wait()
        g_vec0 = g_row[0, pl.ds(0, L)]
        g_vec1 = g_row[0, pl.ds(L, L)]
        iota16b = lax.iota(jnp.int32, L)
        for j in range(D):
            v = (iota16b + j) % D
            lo = jnp.take_along_axis(g_vec0, v % L, axis=0)
            hi = jnp.take_along_axis(g_vec1, v % L, axis=0)
            g_bcast[j] = jnp.where(v < L, lo, hi)

        pltpu.sync_copy(zeros_v, deg_sh.at[pl.ds(sid * u_slice, u_slice)])
        plsc.subcore_barrier()

        def deg_body(i, carry):
            base = (sid + i * NUM_SUBCORES) * C
            pltpu.sync_copy(dst_hbm.at[pl.ds(base, C)], idx_d)
            pltpu.sync_copy(ones_v, deg_sh.at[idx_d], add=True)
            return carry

        n_deg = (n_chunks - sid + NUM_SUBCORES - 1) // NUM_SUBCORES
        lax.fori_loop(0, n_deg, deg_body, 0)

        def p_body(q, carry):
            ubase = sid * u_slice + q * PC
            pltpu.sync_copy(user_hbm.at[pl.ds(ubase, PC)],
                            rows_s.at[pl.ds(0, PC)])

            def p_group(g, gcarry):
                eidx = iota16 + g * L
                acc = zero16
                for j in range(D):
                    jj = (iota16 + j) % D
                    col = plsc.load_gather(rows_s, [eidx, jj])
                    acc = acc + col * g_bcast[j]
                out_e[pl.ds(g * L, L)] = acc
                return gcarry
            lax.fori_loop(0, PC // L, p_group, 0)
            pltpu.sync_copy(out_e.at[pl.ds(0, PC)],
                            p_sh.at[pl.ds(ubase, PC)])
            return carry
        lax.fori_loop(0, n_pc, p_body, 0)
        plsc.subcore_barrier()

        n_edge = (n_chunks - wid + NW - 1) // NW

        def fetch_chunk(i, slot):
            base = (wid + i * NW) * C
            pltpu.sync_copy(src_hbm.at[pl.ds(base, C)], idx_s)
            pltpu.sync_copy(dst_hbm.at[pl.ds(base, C)], idx_d)
            pltpu.sync_copy(deg_sh.at[idx_d], deg_e.at[pl.ds(slot * C, C)])
            pltpu.sync_copy(p_sh.at[idx_s], p_e.at[pl.ds(slot * C, C)])
            cp_s = pltpu.async_copy(
                user_hbm.at[idx_s], rows_s.at[pl.ds(slot * C, C)], sem0)
            cp_d = pltpu.async_copy(
                user_hbm.at[idx_d], rows_d.at[pl.ds(slot * C, C)], sem1)
            return cp_s, cp_d

        def compute_chunk(i, slot):
            def group(g, gcarry):
                off = slot * C + g * L
                eidx = iota16 + off
                acc_st = zero16
                for j in range(D):
                    jj = (iota16 + j) % D
                    s_col = plsc.load_gather(rows_s, [eidx, jj])
                    t_col = plsc.load_gather(rows_d, [eidx, jj])
                    acc_st = acc_st + s_col * t_col
                deg = deg_e[pl.ds(off, L)]
                acc_pf = p_e[pl.ds(off, L)]
                att = _sigmoid(acc_st) / deg
                out_e[pl.ds(g * L, L)] = _sigmoid(att * acc_pf) - 0.5
                return gcarry
            lax.fori_loop(0, C // L, group, 0)
            base = (wid + i * NW) * C
            pltpu.sync_copy(out_e, out_hbm.at[pl.ds(base, C)])

        for d in fetch_chunk(0, 0):
            d.wait()

        def edge_body(i, carry):
            slot = lax.rem(i, 2)
            nslot = lax.rem(i + 1, 2)

            @pl.when(i + 1 < n_edge)
            def _():
                ds_pair = fetch_chunk(i + 1, nslot)
                compute_chunk(i, slot)
                for d in ds_pair:
                    d.wait()

            @pl.when(i + 1 >= n_edge)
            def _():
                compute_chunk(i, slot)
            return carry
        lax.fori_loop(0, n_edge, edge_body, 0)

    return k


def kernel(user_emb, item_emb, u_trust, guided_item):
    U = user_emb.shape[0]
    I = item_emb.shape[0]
    E = u_trust.shape[1]
    src = u_trust[0].astype(jnp.int32)
    dst = u_trust[1].astype(jnp.int32)
    gidx = jnp.full((8,), guided_item, jnp.int32)
    u_slice = ((U + NUM_SUBCORES - 1) // NUM_SUBCORES + PC - 1) // PC * PC
    u_pad = u_slice * NUM_SUBCORES
    user_padded = jnp.pad(user_emb.astype(jnp.float32),
                          ((0, u_pad - U), (0, 0)))
    out = _make_kernel(U, I, E)(
        user_padded, item_emb.astype(jnp.float32), src, dst, gidx)
    return out.reshape(E, 1)


# speedup vs baseline: 24.2081x; 1.0526x over previous
import functools

import jax
import jax.numpy as jnp
from jax import lax
from jax.experimental import pallas as pl
from jax.experimental.pallas import tpu as pltpu
from jax.experimental.pallas import tpu_sc as plsc

NUM_CORES = 2
NUM_SUBCORES = 16
NW = NUM_CORES * NUM_SUBCORES
L = 16
D = 32
C = 640
PC = 448


def _sigmoid(x):
    return 1.0 / (1.0 + jnp.exp(-x))


def _make_kernel(U, I, E):
    assert E % C == 0
    n_chunks = E // C
    u_slice = ((U + NUM_SUBCORES - 1) // NUM_SUBCORES + PC - 1) // PC * PC
    n_pc = u_slice // PC
    u_pad = u_slice * NUM_SUBCORES

    mesh = plsc.VectorSubcoreMesh(
        core_axis_name="c", subcore_axis_name="s",
        num_cores=NUM_CORES, num_subcores=NUM_SUBCORES)

    @functools.partial(
        pl.kernel,
        out_type=jax.ShapeDtypeStruct((E,), jnp.float32),
        mesh=mesh,
        scratch_types=[
            pltpu.VMEM((C,), jnp.int32),
            pltpu.VMEM((C,), jnp.int32),
            pltpu.VMEM((2 * C, D), jnp.float32),
            pltpu.VMEM((2 * C, D), jnp.float32),
            pltpu.VMEM((2 * C,), jnp.float32),
            pltpu.VMEM((2 * C,), jnp.float32),
            pltpu.VMEM((C,), jnp.float32),
            pltpu.VMEM((C,), jnp.float32),
            pltpu.VMEM((u_slice,), jnp.float32),
            pltpu.VMEM((1, D), jnp.float32),
            pltpu.VMEM((D, L), jnp.float32),
            pltpu.VMEM((1,), jnp.int32),
            pltpu.VMEM_SHARED((u_pad,), jnp.float32),
            pltpu.VMEM_SHARED((u_pad,), jnp.float32),
            pltpu.SemaphoreType.DMA,
            pltpu.SemaphoreType.DMA,
        ],
        compiler_params=pltpu.CompilerParams(
            needs_layout_passes=False, use_tc_tiling_on_sc=False),
    )
    def k(user_hbm, item_hbm, src_hbm, dst_hbm, gidx_hbm, out_hbm,
          idx_s, idx_d, rows_s, rows_d, deg_e, p_e, out_e, ones_v, zeros_v,
          g_row, g_bcast, g_idx_v, deg_sh, p_sh, sem0, sem1):
        cid = lax.axis_index("c")
        sid = lax.axis_index("s")
        wid = sid * NUM_CORES + cid

        zero16 = jnp.zeros((L,), jnp.float32)
        iota16 = lax.iota(jnp.int32, L)
        one16 = jnp.ones((L,), jnp.float32)

        def fill_zeros(i, carry):
            zeros_v[pl.ds(i * L, L)] = zero16
            return carry

        lax.fori_loop(0, u_slice // L, fill_zeros, 0)

        def fill_ones(i, carry):
            ones_v[pl.ds(i * L, L)] = one16
            return carry

        lax.fori_loop(0, C // L, fill_ones, 0)

        pltpu.sync_copy(gidx_hbm.at[pl.ds(0, 1)], g_idx_v)
        pltpu.async_copy(item_hbm.at[g_idx_v], g_row, sem0).wait()
        g_vec0 = g_row[0, pl.ds(0, L)]
        g_vec1 = g_row[0, pl.ds(L, L)]
        iota16b = lax.iota(jnp.int32, L)
        for j in range(D):
            v = (iota16b + j) % D
            lo = jnp.take_along_axis(g_vec0, v % L, axis=0)
            hi = jnp.take_along_axis(g_vec1, v % L, axis=0)
            g_bcast[j] = jnp.where(v < L, lo, hi)

        pltpu.sync_copy(zeros_v, deg_sh.at[pl.ds(sid * u_slice, u_slice)])
        plsc.subcore_barrier()

        def deg_body(i, carry):
            base = (sid + i * NUM_SUBCORES) * C
            pltpu.sync_copy(dst_hbm.at[pl.ds(base, C)], idx_d)
            pltpu.sync_copy(ones_v, deg_sh.at[idx_d], add=True)
            return carry

        n_deg = (n_chunks - sid + NUM_SUBCORES - 1) // NUM_SUBCORES
        lax.fori_loop(0, n_deg, deg_body, 0)

        def p_body(q, carry):
            ubase = sid * u_slice + q * PC
            pltpu.sync_copy(user_hbm.at[pl.ds(ubase, PC)],
                            rows_s.at[pl.ds(0, PC)])

            def p_group(g, gcarry):
                eidx = iota16 + g * L
                acc = zero16
                for j in range(D):
                    jj = (iota16 + j) % D
                    col = plsc.load_gather(rows_s, [eidx, jj])
                    acc = acc + col * g_bcast[j]
                out_e[pl.ds(g * L, L)] = acc
                return gcarry
            lax.fori_loop(0, PC // L, p_group, 0)
            pltpu.sync_copy(out_e.at[pl.ds(0, PC)],
                            p_sh.at[pl.ds(ubase, PC)])
            return carry
        lax.fori_loop(0, n_pc, p_body, 0)
        plsc.subcore_barrier()

        n_edge = (n_chunks - wid + NW - 1) // NW

        def fetch_chunk(i, slot):
            base = (wid + i * NW) * C
            pltpu.sync_copy(src_hbm.at[pl.ds(base, C)], idx_s)
            pltpu.sync_copy(dst_hbm.at[pl.ds(base, C)], idx_d)
            pltpu.sync_copy(deg_sh.at[idx_d], deg_e.at[pl.ds(slot * C, C)])
            pltpu.sync_copy(p_sh.at[idx_s], p_e.at[pl.ds(slot * C, C)])
            cp_s = pltpu.async_copy(
                user_hbm.at[idx_s], rows_s.at[pl.ds(slot * C, C)], sem0)
            cp_d = pltpu.async_copy(
                user_hbm.at[idx_d], rows_d.at[pl.ds(slot * C, C)], sem1)
            return cp_s, cp_d

        def compute_chunk(i, slot):
            def group(g, gcarry):
                off = slot * C + g * L
                eidx = iota16 + off
                acc_st = zero16
                for j in range(D):
                    jj = (iota16 + j) % D
                    s_col = plsc.load_gather(rows_s, [eidx, jj])
                    t_col = plsc.load_gather(rows_d, [eidx, jj])
                    acc_st = acc_st + s_col * t_col
                deg = deg_e[pl.ds(off, L)]
                acc_pf = p_e[pl.ds(off, L)]
                att = _sigmoid(acc_st) / deg
                out_e[pl.ds(g * L, L)] = _sigmoid(att * acc_pf) - 0.5
                return gcarry
            lax.fori_loop(0, C // L, group, 0)
            base = (wid + i * NW) * C
            pltpu.sync_copy(out_e, out_hbm.at[pl.ds(base, C)])

        for d in fetch_chunk(0, 0):
            d.wait()

        def edge_body(i, carry):
            slot = lax.rem(i, 2)
            nslot = lax.rem(i + 1, 2)

            @pl.when(i + 1 < n_edge)
            def _():
                ds_next = fetch_chunk(i + 1, nslot)
                compute_chunk(i, slot)
                for d in ds_next:
                    d.wait()

            @pl.when(i + 1 >= n_edge)
            def _():
                compute_chunk(i, slot)
            return carry
        lax.fori_loop(0, n_edge, edge_body, 0)

    return k


def kernel(user_emb, item_emb, u_trust, guided_item):
    U = user_emb.shape[0]
    I = item_emb.shape[0]
    E = u_trust.shape[1]
    src = u_trust[0].astype(jnp.int32)
    dst = u_trust[1].astype(jnp.int32)
    gidx = jnp.full((8,), guided_item, jnp.int32)
    u_slice = ((U + NUM_SUBCORES - 1) // NUM_SUBCORES + PC - 1) // PC * PC
    u_pad = u_slice * NUM_SUBCORES
    user_padded = jnp.pad(user_emb.astype(jnp.float32),
                          ((0, u_pad - U), (0, 0)))
    out = _make_kernel(U, I, E)(
        user_padded, item_emb.astype(jnp.float32), src, dst, gidx)
    return out.reshape(E, 1)


# speedup vs baseline: 30.7495x; 1.2702x over previous
import functools

import jax
import jax.numpy as jnp
from jax import lax
from jax.experimental import pallas as pl
from jax.experimental.pallas import tpu as pltpu
from jax.experimental.pallas import tpu_sc as plsc

NUM_CORES = 2
NUM_SUBCORES = 16
NW = NUM_CORES * NUM_SUBCORES
L = 16
D = 32
C = 640
PC = 448


def _sigmoid(x):
    return 1.0 / (1.0 + jnp.exp(-x))


def _make_kernel(U, I, E):
    assert E % C == 0
    n_chunks = E // C
    u_slice = ((U + NUM_SUBCORES - 1) // NUM_SUBCORES + PC - 1) // PC * PC
    n_pc = u_slice // PC
    u_pad = u_slice * NUM_SUBCORES
    assert n_pc % 2 == 0 and PC <= C

    mesh = plsc.VectorSubcoreMesh(
        core_axis_name="c", subcore_axis_name="s",
        num_cores=NUM_CORES, num_subcores=NUM_SUBCORES)

    @functools.partial(
        pl.kernel,
        out_type=jax.ShapeDtypeStruct((E,), jnp.float32),
        mesh=mesh,
        scratch_types=[
            pltpu.VMEM((C,), jnp.int32),
            pltpu.VMEM((C,), jnp.int32),
            pltpu.VMEM((C,), jnp.int32),
            pltpu.VMEM((C,), jnp.int32),
            pltpu.VMEM((2 * C, D), jnp.float32),
            pltpu.VMEM((2 * C, D), jnp.float32),
            pltpu.VMEM((2 * C,), jnp.float32),
            pltpu.VMEM((2 * C,), jnp.float32),
            pltpu.VMEM((C,), jnp.float32),
            pltpu.VMEM((C,), jnp.float32),
            pltpu.VMEM((u_slice,), jnp.float32),
            pltpu.VMEM((1, D), jnp.float32),
            pltpu.VMEM((D, L), jnp.float32),
            pltpu.VMEM((1,), jnp.int32),
            pltpu.VMEM_SHARED((u_pad,), jnp.float32),
            pltpu.VMEM_SHARED((u_pad,), jnp.float32),
            pltpu.SemaphoreType.DMA,
            pltpu.SemaphoreType.DMA,
        ],
        compiler_params=pltpu.CompilerParams(
            needs_layout_passes=False, use_tc_tiling_on_sc=False),
    )
    def k(user_hbm, item_hbm, src_hbm, dst_hbm, gidx_hbm, out_hbm,
          idx_s, idx_d, idx_s2, idx_d2, rows_s, rows_d, deg_e, p_e, out_e,
          ones_v, zeros_v, g_row, g_bcast, g_idx_v, deg_sh, p_sh,
          sem0, sem1):
        cid = lax.axis_index("c")
        sid = lax.axis_index("s")
        wid = sid * NUM_CORES + cid

        zero16 = jnp.zeros((L,), jnp.float32)
        iota16 = lax.iota(jnp.int32, L)
        one16 = jnp.ones((L,), jnp.float32)

        def fill_zeros(i, carry):
            zeros_v[pl.ds(i * L, L)] = zero16
            return carry

        lax.fori_loop(0, u_slice // L, fill_zeros, 0)

        def fill_ones(i, carry):
            ones_v[pl.ds(i * L, L)] = one16
            return carry

        lax.fori_loop(0, C // L, fill_ones, 0)

        pltpu.sync_copy(gidx_hbm.at[pl.ds(0, 1)], g_idx_v)
        pltpu.async_copy(item_hbm.at[g_idx_v], g_row, sem0).wait()
        g_vec0 = g_row[0, pl.ds(0, L)]
        g_vec1 = g_row[0, pl.ds(L, L)]
        iota16b = lax.iota(jnp.int32, L)
        for j in range(D):
            v = (iota16b + j) % D
            lo = jnp.take_along_axis(g_vec0, v % L, axis=0)
            hi = jnp.take_along_axis(g_vec1, v % L, axis=0)
            g_bcast[j] = jnp.where(v < L, lo, hi)

        pltpu.sync_copy(zeros_v, deg_sh.at[pl.ds(sid * u_slice, u_slice)])
        plsc.subcore_barrier()

        n_deg = (n_chunks - sid + NUM_SUBCORES - 1) // NUM_SUBCORES

        def deg_fetch(i, ref):
            base = (sid + i * NUM_SUBCORES) * C
            return pltpu.async_copy(dst_hbm.at[pl.ds(base, C)], ref, sem0)

        deg_fetch(0, idx_d).wait()

        def deg_pair(kk, carry):
            a = 2 * kk
            b = a + 1

            @pl.when(b < n_deg)
            def _():
                d = deg_fetch(b, idx_d2)
                pltpu.sync_copy(ones_v, deg_sh.at[idx_d], add=True)
                d.wait()

                @pl.when(b + 1 < n_deg)
                def _():
                    d2 = deg_fetch(b + 1, idx_d)
                    pltpu.sync_copy(ones_v, deg_sh.at[idx_d2], add=True)
                    d2.wait()

                @pl.when(b + 1 >= n_deg)
                def _():
                    pltpu.sync_copy(ones_v, deg_sh.at[idx_d2], add=True)

            @pl.when(b >= n_deg)
            def _():
                pltpu.sync_copy(ones_v, deg_sh.at[idx_d], add=True)
            return carry
        lax.fori_loop(0, (n_deg + 1) // 2, deg_pair, 0)

        def p_rows_copy(q, slot):
            ubase = sid * u_slice + q * PC
            return pltpu.async_copy(
                user_hbm.at[pl.ds(ubase, PC)],
                rows_s.at[pl.ds(slot * C, PC)], sem1)

        def p_step(q, slot):
            def p_group(g, gcarry):
                off = slot * C + g * L
                eidx = iota16 + off
                acc = zero16
                for j in range(D):
                    jj = (iota16 + j) % D
                    col = plsc.load_gather(rows_s, [eidx, jj])
                    acc = acc + col * g_bcast[j]
                out_e[pl.ds(g * L, L)] = acc
                return gcarry
            lax.fori_loop(0, PC // L, p_group, 0)
            pltpu.sync_copy(
                out_e.at[pl.ds(0, PC)],
                p_sh.at[pl.ds(sid * u_slice + q * PC, PC)])

        p_rows_copy(0, 0).wait()

        def p_pair(kk, carry):
            qa = 2 * kk
            qb = qa + 1
            d_b = p_rows_copy(qb, 1)
            p_step(qa, 0)
            d_b.wait()

            @pl.when(qb + 1 < n_pc)
            def _():
                d_a = p_rows_copy(qb + 1, 0)
                p_step(qb, 1)
                d_a.wait()

            @pl.when(qb + 1 >= n_pc)
            def _():
                p_step(qb, 1)
            return carry
        lax.fori_loop(0, n_pc // 2, p_pair, 0)
        plsc.subcore_barrier()

        n_edge = (n_chunks - wid + NW - 1) // NW

        def idx_fetch(i, ref_s, ref_d):
            base = (wid + i * NW) * C
            d1 = pltpu.async_copy(src_hbm.at[pl.ds(base, C)], ref_s, sem0)
            d2 = pltpu.async_copy(dst_hbm.at[pl.ds(base, C)], ref_d, sem0)
            return d1, d2

        def gather_issue(ref_s, ref_d, slot):
            pltpu.sync_copy(deg_sh.at[ref_d], deg_e.at[pl.ds(slot * C, C)])
            pltpu.sync_copy(p_sh.at[ref_s], p_e.at[pl.ds(slot * C, C)])
            d1 = pltpu.async_copy(
                user_hbm.at[ref_s], rows_s.at[pl.ds(slot * C, C)], sem0)
            d2 = pltpu.async_copy(
                user_hbm.at[ref_d], rows_d.at[pl.ds(slot * C, C)], sem1)
            return d1, d2

        def compute_chunk(i, slot):
            def group(g, gcarry):
                off = slot * C + g * L
                eidx = iota16 + off
                acc_st = zero16
                for j in range(D):
                    jj = (iota16 + j) % D
                    s_col = plsc.load_gather(rows_s, [eidx, jj])
                    t_col = plsc.load_gather(rows_d, [eidx, jj])
                    acc_st = acc_st + s_col * t_col
                deg = deg_e[pl.ds(off, L)]
                acc_pf = p_e[pl.ds(off, L)]
                att = _sigmoid(acc_st) / deg
                out_e[pl.ds(g * L, L)] = _sigmoid(att * acc_pf) - 0.5
                return gcarry
            lax.fori_loop(0, C // L, group, 0)
            base = (wid + i * NW) * C
            pltpu.sync_copy(out_e, out_hbm.at[pl.ds(base, C)])

        for d in idx_fetch(0, idx_s, idx_d):
            d.wait()
        g0 = gather_issue(idx_s, idx_d, 0)

        @pl.when(1 < n_edge)
        def _():
            for d in idx_fetch(1, idx_s2, idx_d2):
                d.wait()
        for d in g0:
            d.wait()

        def edge_pair(kk, carry):
            a = 2 * kk
            b = a + 1

            @pl.when(b < n_edge)
            def _():
                dg_b = gather_issue(idx_s2, idx_d2, 1)

                @pl.when(a + 2 < n_edge)
                def _():
                    di_a = idx_fetch(a + 2, idx_s, idx_d)
                    compute_chunk(a, 0)
                    for d in di_a:
                        d.wait()

                @pl.when(a + 2 >= n_edge)
                def _():
                    compute_chunk(a, 0)

                for d in dg_b:
                    d.wait()

                @pl.when(a + 2 < n_edge)
                def _():
                    dg_a = gather_issue(idx_s, idx_d, 0)

                    @pl.when(b + 2 < n_edge)
                    def _():
                        di_b = idx_fetch(b + 2, idx_s2, idx_d2)
                        compute_chunk(b, 1)
                        for d in di_b:
                            d.wait()

                    @pl.when(b + 2 >= n_edge)
                    def _():
                        compute_chunk(b, 1)

                    for d in dg_a:
                        d.wait()

                @pl.when(a + 2 >= n_edge)
                def _():
                    compute_chunk(b, 1)

            @pl.when(b >= n_edge)
            def _():
                compute_chunk(a, 0)
            return carry
        lax.fori_loop(0, (n_edge + 1) // 2, edge_pair, 0)

    return k


def kernel(user_emb, item_emb, u_trust, guided_item):
    U = user_emb.shape[0]
    I = item_emb.shape[0]
    E = u_trust.shape[1]
    src = u_trust[0].astype(jnp.int32)
    dst = u_trust[1].astype(jnp.int32)
    gidx = jnp.full((8,), guided_item, jnp.int32)
    u_slice = ((U + NUM_SUBCORES - 1) // NUM_SUBCORES + PC - 1) // PC * PC
    u_pad = u_slice * NUM_SUBCORES
    user_padded = jnp.pad(user_emb.astype(jnp.float32),
                          ((0, u_pad - U), (0, 0)))
    out = _make_kernel(U, I, E)(
        user_padded, item_emb.astype(jnp.float32), src, dst, gidx)
    return out.reshape(E, 1)
